# baseline (speedup 1.0000x reference)
import functools

import jax
import jax.numpy as jnp
from jax import lax
from jax.experimental import pallas as pl
from jax.experimental.pallas import tpu as pltpu

EPSILON = 1e-15
LN_EPS = 1e-12
B, S_FULL, S, D = 2, 2048, 512, 1024
G = 8
T = 256
NT = S_FULL // T
TE = 128
NE = S_FULL // TE


def _gather_body(idx_ref, *refs):
    out_ref = refs[G]
    b = pl.program_id(0)
    j = pl.program_id(1)
    rows = []
    for g in range(G):
        r = idx_ref[b * S + j * G + g] % 8
        rows.append(refs[g][0, pl.ds(r, 1), :])
    out_ref[0] = jnp.concatenate(rows, axis=0)


def _gather_rows(hidden_states, flat_loc):
    grid = (B, S // G)
    in_specs = [
        pl.BlockSpec((1, 8, D), functools.partial(
            lambda g, b, j, idx: (b, idx[b * S + j * G + g] // 8, 0), g))
        for g in range(G)
    ]
    out_spec = pl.BlockSpec((1, G, D), lambda b, j, idx: (b, j, 0))
    return pl.pallas_call(
        _gather_body,
        grid_spec=pltpu.PrefetchScalarGridSpec(
            num_scalar_prefetch=1,
            grid=grid,
            in_specs=in_specs,
            out_specs=out_spec,
        ),
        out_shape=jax.ShapeDtypeStruct((B, S, D), jnp.float32),
    )(flat_loc, *([hidden_states] * G))


def _qk_body(ctx_ref, wq_ref, bq_ref, wk_ref, bk_ref, lnw_ref, lnb_ref, out_ref):
    x = ctx_ref[0]
    mu = jnp.mean(x, axis=1, keepdims=True)
    var = jnp.mean((x - mu) ** 2, axis=1, keepdims=True)
    ctx = (x - mu) / jnp.sqrt(var + LN_EPS) * lnw_ref[...] + lnb_ref[...]
    q = lax.dot_general(ctx, wq_ref[...], (((1,), (1,)), ((), ())),
                        preferred_element_type=jnp.float32) + bq_ref[...]
    k = lax.dot_general(ctx, wk_ref[...], (((1,), (1,)), ((), ())),
                        preferred_element_type=jnp.float32) + bk_ref[...]
    zrow = jnp.zeros((1, D), jnp.float32)
    k_next = jnp.concatenate([k[1:], zrow], axis=0)
    q_next = jnp.concatenate([q[1:], zrow], axis=0)
    scale = D / 2.0
    f = jnp.sum(q * k_next, axis=1, keepdims=True) / scale
    g = jnp.sum(q_next * k, axis=1, keepdims=True) / scale
    zc = jnp.zeros((1, 1), jnp.float32)
    g_prev = jnp.concatenate([zc, g[:-1]], axis=0)
    f_next = jnp.concatenate([f[1:], zc], axis=0)

    def two_sm(a, b):
        m = jnp.maximum(a, b)
        ea = jnp.exp(a - m)
        eb = jnp.exp(b - m)
        return ea / (ea + eb)

    i_col = lax.broadcasted_iota(jnp.int32, (S, 1), 0)
    p = jnp.where(i_col == 0, 1.0, two_sm(f, g_prev))
    r = jnp.where(i_col == S - 2, 1.0, two_sm(g, f_next))
    a_diag = jnp.sqrt(p * r + EPSILON)
    out_ref[0] = jnp.where(i_col <= S - 2, a_diag, 0.0)


def _qk_adiag(ctx, Wq, bq, Wk, bk, ln_w, ln_b):
    return pl.pallas_call(
        _qk_body,
        grid=(B,),
        in_specs=[
            pl.BlockSpec((1, S, D), lambda b: (b, 0, 0)),
            pl.BlockSpec((D, D), lambda b: (0, 0)),
            pl.BlockSpec((1, D), lambda b: (0, 0)),
            pl.BlockSpec((D, D), lambda b: (0, 0)),
            pl.BlockSpec((1, D), lambda b: (0, 0)),
            pl.BlockSpec((1, D), lambda b: (0, 0)),
            pl.BlockSpec((1, D), lambda b: (0, 0)),
        ],
        out_specs=pl.BlockSpec((1, S, 1), lambda b: (b, 0, 0)),
        out_shape=jax.ShapeDtypeStruct((B, S, 1), jnp.float32),
    )(ctx, Wq, bq.reshape(1, D), Wk, bk.reshape(1, D),
      ln_w.reshape(1, D), ln_b.reshape(1, D))


def _sup_body(diag_ref, right_ref, out_ref):
    a = diag_ref[0]
    bmat = right_ref[0]
    lr = lax.broadcasted_iota(jnp.int32, (TE, TE), 0)
    lc = lax.broadcasted_iota(jnp.int32, (TE, TE), 1)
    u1 = jnp.sum(jnp.where(lr == lc - 1, a, 0.0), axis=0)
    u1 = u1.reshape(1, TE)
    u1s = jnp.concatenate([u1[:, 1:], jnp.zeros((1, 1), jnp.float32)], axis=1)
    u2 = jnp.sum(jnp.where(lr == TE - 1, bmat, 0.0), axis=0).reshape(1, TE)
    u2r = jnp.concatenate([jnp.zeros((1, TE - 1), jnp.float32), u2[:, :1]],
                          axis=1)
    out_ref[0] = u1s + u2r


def _prior_sup(prior_A):
    return pl.pallas_call(
        _sup_body,
        grid=(B, NE),
        in_specs=[
            pl.BlockSpec((1, TE, TE), lambda b, r: (b, r, r)),
            pl.BlockSpec((1, TE, TE),
                         lambda b, r: (b, r, jnp.minimum(r + 1, NE - 1))),
        ],
        out_specs=pl.BlockSpec((1, 1, TE), lambda b, r: (b, 0, r)),
        out_shape=jax.ShapeDtypeStruct((B, 1, S_FULL), jnp.float32),
    )(prior_A, prior_A)


def _assemble_body(adiag_ref, loc_ref, amask_ref, psup_ref,
                   wl_ref, wr_ref, pl_ref, pr_ref):
    adiag = adiag_ref[0]
    loc = loc_ref[0]
    i_col = lax.broadcasted_iota(jnp.int32, (S, 1), 0)
    loc_next = jnp.concatenate([loc[1:], loc[-1:]], axis=0)
    keep = ((loc != loc_next) | (i_col == S - 2)) & (i_col <= S - 2)
    j_row = lax.broadcasted_iota(jnp.int32, (S, S_FULL), 1)
    cmp = (loc == j_row) & keep
    val = jnp.sum(jnp.where(cmp, adiag, 0.0), axis=0).reshape(1, S_FULL)
    hit = jnp.max(jnp.where(cmp, 1.0, 0.0), axis=0).reshape(1, S_FULL)
    w = jnp.where(hit > 0.0, val, amask_ref[0])
    psup = psup_ref[0]
    a_sup = psup + (1.0 - psup) * w
    lane = lax.broadcasted_iota(jnp.int32, (1, S_FULL), 1)
    t = jnp.where(lane <= S_FULL - 2, jnp.log(a_sup + EPSILON), 0.0)
    sh = 1
    while sh < S_FULL:
        t = t + jnp.concatenate(
            [jnp.zeros((1, sh), jnp.float32), t[:, : S_FULL - sh]], axis=1)
        sh *= 2
    p = jnp.concatenate(
        [jnp.zeros((1, 1), jnp.float32), t[:, : S_FULL - 1]], axis=1)
    wl_ref[0] = w
    pl_ref[0] = p
    wr_ref[0] = jnp.transpose(w)
    pr_ref[0] = jnp.transpose(p)


def _assemble(adiag, loc_col, amask3, psup3):
    return pl.pallas_call(
        _assemble_body,
        grid=(B,),
        in_specs=[
            pl.BlockSpec((1, S, 1), lambda b: (b, 0, 0)),
            pl.BlockSpec((1, S, 1), lambda b: (b, 0, 0)),
            pl.BlockSpec((1, 1, S_FULL), lambda b: (b, 0, 0)),
            pl.BlockSpec((1, 1, S_FULL), lambda b: (b, 0, 0)),
        ],
        out_specs=[
            pl.BlockSpec((1, 1, S_FULL), lambda b: (b, 0, 0)),
            pl.BlockSpec((1, S_FULL, 1), lambda b: (b, 0, 0)),
            pl.BlockSpec((1, 1, S_FULL), lambda b: (b, 0, 0)),
            pl.BlockSpec((1, S_FULL, 1), lambda b: (b, 0, 0)),
        ],
        out_shape=[
            jax.ShapeDtypeStruct((B, 1, S_FULL), jnp.float32),
            jax.ShapeDtypeStruct((B, S_FULL, 1), jnp.float32),
            jax.ShapeDtypeStruct((B, 1, S_FULL), jnp.float32),
            jax.ShapeDtypeStruct((B, S_FULL, 1), jnp.float32),
        ],
    )(adiag, loc_col, amask3, psup3)


def _big_body(prior_ref, wr_ref, wc_ref, pr_ref, pc_ref, anew_ref, c_ref):
    prior = prior_ref[0]
    r = pl.program_id(1)
    c = pl.program_id(2)
    p_row = pr_ref[0]
    p_col = pc_ref[0, 0].reshape(1, T)

    @pl.when(r == c)
    def _diag():
        row = lax.broadcasted_iota(jnp.int32, (T, T), 0)
        col = lax.broadcasted_iota(jnp.int32, (T, T), 1)
        w_row = wr_ref[0]
        w_col = wc_ref[0, 0].reshape(1, T)
        m = jnp.where(col == row + 1, w_row,
                      jnp.where(row == col + 1, w_col, 0.0))
        anew_ref[0] = prior + (1.0 - prior) * m
        delta = jnp.where(col >= row, p_col - p_row, p_row - p_col)
        c_ref[0] = jnp.exp(delta)

    @pl.when(r != c)
    def _offdiag():
        sgn = jnp.where(c > r, 1.0, -1.0)
        c_ref[0] = jnp.exp((p_col - p_row) * sgn)
        anew_ref[0] = prior

        lane = lax.broadcasted_iota(jnp.int32, (1, T), 1)

        @pl.when(c == r + 1)
        def _fix_up():
            prow = prior[T - 1:T, :]
            wv = wr_ref[0][T - 1:T, :]
            fixed = prow + (1.0 - prow) * wv
            anew_ref[0, T - 1:T, :] = jnp.where(lane == 0, fixed, prow)

        @pl.when(c == r - 1)
        def _fix_lo():
            prow = prior[0:1, :]
            wv = wc_ref[0][:, T - 1:T]
            fixed = prow + (1.0 - prow) * wv
            anew_ref[0, 0:1, :] = jnp.where(lane == T - 1, fixed, prow)


def _big_outputs(prior_A, w_lane, w_rowv, p_lane, p_rowv):
    return pl.pallas_call(
        _big_body,
        grid=(B, NT, NT),
        in_specs=[
            pl.BlockSpec((1, T, T), lambda b, r, c: (b, r, c)),
            pl.BlockSpec((1, T, 1), lambda b, r, c: (b, r, 0)),
            pl.BlockSpec((1, 1, T), lambda b, r, c: (b, 0, c)),
            pl.BlockSpec((1, T, 1), lambda b, r, c: (b, r, 0)),
            pl.BlockSpec((1, 1, T), lambda b, r, c: (b, 0, c)),
        ],
        out_specs=[
            pl.BlockSpec((1, T, T), lambda b, r, c: (b, r, c)),
            pl.BlockSpec((1, T, T), lambda b, r, c: (b, r, c)),
        ],
        out_shape=[
            jax.ShapeDtypeStruct((B, S_FULL, S_FULL), jnp.float32),
            jax.ShapeDtypeStruct((B, S_FULL, S_FULL), jnp.float32),
        ],
        compiler_params=pltpu.CompilerParams(
            dimension_semantics=("parallel", "parallel", "parallel")),
    )(prior_A, w_rowv, w_lane, p_rowv, p_lane)


def kernel(hidden_states, attention_mask, ip_event_loc, ip_event_mask,
           prior_A, Wk, bk, Wq, bq, ln_w, ln_b):
    loc = ip_event_loc.astype(jnp.int32)
    flat_loc = loc.reshape(-1)

    gathered = _gather_rows(hidden_states, flat_loc)
    adiag = _qk_adiag(gathered, Wq, bq, Wk, bk, ln_w, ln_b)
    psup3 = _prior_sup(prior_A)
    amask3 = attention_mask.astype(jnp.float32).reshape(B, 1, S_FULL)
    loc_col = loc.reshape(B, S, 1)
    w_lane, w_rowv, p_lane, p_rowv = _assemble(
        adiag, loc_col, amask3, psup3)
    anew, c_prior = _big_outputs(prior_A, w_lane, w_rowv, p_lane, p_rowv)
    return (c_prior.astype(jnp.float32), anew.astype(jnp.float32))


_PROBE = 0

if _PROBE == 1:
    _full = kernel
    def kernel(hidden_states, attention_mask, ip_event_loc, ip_event_mask,
               prior_A, Wk, bk, Wq, bq, ln_w, ln_b):
        z_lane = jnp.zeros((B, 1, S_FULL), jnp.float32)
        z_row = jnp.zeros((B, S_FULL, 1), jnp.float32)
        anew, c_prior = _big_outputs(prior_A, z_lane, z_row, z_lane, z_row)
        return (c_prior, anew)
elif _PROBE == 2:
    _full = kernel
    def kernel(hidden_states, attention_mask, ip_event_loc, ip_event_mask,
               prior_A, Wk, bk, Wq, bq, ln_w, ln_b):
        loc = ip_event_loc.astype(jnp.int32)
        flat_loc = loc.reshape(-1)
        gathered = _gather_rows(hidden_states, flat_loc)
        adiag = _qk_adiag(gathered, Wq, bq, Wk, bk, ln_w, ln_b)
        psup3 = _prior_sup(prior_A)
        amask3 = attention_mask.astype(jnp.float32).reshape(B, 1, S_FULL)
        loc_col = loc.reshape(B, S, 1)
        w_lane, w_rowv, p_lane, p_rowv = _assemble(adiag, loc_col, amask3, psup3)
        return (w_lane, p_lane)
elif _PROBE == 3:
    _full = kernel
    def kernel(hidden_states, attention_mask, ip_event_loc, ip_event_mask,
               prior_A, Wk, bk, Wq, bq, ln_w, ln_b):
        loc = ip_event_loc.astype(jnp.int32)
        flat_loc = loc.reshape(-1)
        gathered = _gather_rows(hidden_states, flat_loc)
        adiag = _qk_adiag(gathered, Wq, bq, Wk, bk, ln_w, ln_b)
        return (adiag,)
elif _PROBE == 4:
    _full = kernel
    def kernel(hidden_states, attention_mask, ip_event_loc, ip_event_mask,
               prior_A, Wk, bk, Wq, bq, ln_w, ln_b):
        loc = ip_event_loc.astype(jnp.int32)
        flat_loc = loc.reshape(-1)
        gathered = _gather_rows(hidden_states, flat_loc)
        return (gathered,)


# speedup vs baseline: 1.2939x; 1.2939x over previous
import functools

import jax
import jax.numpy as jnp
from jax import lax
from jax.experimental import pallas as pl
from jax.experimental.pallas import tpu as pltpu

EPSILON = 1e-15
LN_EPS = 1e-12
B, S_FULL, S, D = 2, 2048, 512, 1024
G = 8
T = 512
NT = S_FULL // T
TE = 128
NE = S_FULL // TE


def _gather_body(idx_ref, *refs):
    out_ref = refs[G]
    b = pl.program_id(0)
    j = pl.program_id(1)
    rows = []
    for g in range(G):
        r = idx_ref[b * S + j * G + g] % 8
        rows.append(refs[g][0, pl.ds(r, 1), :])
    out_ref[0] = jnp.concatenate(rows, axis=0)


def _gather_rows(hidden_states, flat_loc):
    grid = (B, S // G)
    in_specs = [
        pl.BlockSpec((1, 8, D), functools.partial(
            lambda g, b, j, idx: (b, idx[b * S + j * G + g] // 8, 0), g))
        for g in range(G)
    ]
    out_spec = pl.BlockSpec((1, G, D), lambda b, j, idx: (b, j, 0))
    return pl.pallas_call(
        _gather_body,
        grid_spec=pltpu.PrefetchScalarGridSpec(
            num_scalar_prefetch=1,
            grid=grid,
            in_specs=in_specs,
            out_specs=out_spec,
        ),
        out_shape=jax.ShapeDtypeStruct((B, S, D), jnp.float32),
    )(flat_loc, *([hidden_states] * G))


def _qk_body(ctx_ref, wq_ref, bq_ref, wk_ref, bk_ref, lnw_ref, lnb_ref, out_ref):
    x = ctx_ref[0]
    mu = jnp.mean(x, axis=1, keepdims=True)
    var = jnp.mean((x - mu) ** 2, axis=1, keepdims=True)
    ctx = (x - mu) / jnp.sqrt(var + LN_EPS) * lnw_ref[...] + lnb_ref[...]
    q = lax.dot_general(ctx, wq_ref[...], (((1,), (1,)), ((), ())),
                        preferred_element_type=jnp.float32) + bq_ref[...]
    k = lax.dot_general(ctx, wk_ref[...], (((1,), (1,)), ((), ())),
                        preferred_element_type=jnp.float32) + bk_ref[...]
    zrow = jnp.zeros((1, D), jnp.float32)
    k_next = jnp.concatenate([k[1:], zrow], axis=0)
    q_next = jnp.concatenate([q[1:], zrow], axis=0)
    scale = D / 2.0
    f = jnp.sum(q * k_next, axis=1, keepdims=True) / scale
    g = jnp.sum(q_next * k, axis=1, keepdims=True) / scale
    zc = jnp.zeros((1, 1), jnp.float32)
    g_prev = jnp.concatenate([zc, g[:-1]], axis=0)
    f_next = jnp.concatenate([f[1:], zc], axis=0)

    def two_sm(a, b):
        m = jnp.maximum(a, b)
        ea = jnp.exp(a - m)
        eb = jnp.exp(b - m)
        return ea / (ea + eb)

    i_col = lax.broadcasted_iota(jnp.int32, (S, 1), 0)
    p = jnp.where(i_col == 0, 1.0, two_sm(f, g_prev))
    r = jnp.where(i_col == S - 2, 1.0, two_sm(g, f_next))
    a_diag = jnp.sqrt(p * r + EPSILON)
    out_ref[0] = jnp.where(i_col <= S - 2, a_diag, 0.0)


def _qk_adiag(ctx, Wq, bq, Wk, bk, ln_w, ln_b):
    return pl.pallas_call(
        _qk_body,
        grid=(B,),
        in_specs=[
            pl.BlockSpec((1, S, D), lambda b: (b, 0, 0)),
            pl.BlockSpec((D, D), lambda b: (0, 0)),
            pl.BlockSpec((1, D), lambda b: (0, 0)),
            pl.BlockSpec((D, D), lambda b: (0, 0)),
            pl.BlockSpec((1, D), lambda b: (0, 0)),
            pl.BlockSpec((1, D), lambda b: (0, 0)),
            pl.BlockSpec((1, D), lambda b: (0, 0)),
        ],
        out_specs=pl.BlockSpec((1, S, 1), lambda b: (b, 0, 0)),
        out_shape=jax.ShapeDtypeStruct((B, S, 1), jnp.float32),
    )(ctx, Wq, bq.reshape(1, D), Wk, bk.reshape(1, D),
      ln_w.reshape(1, D), ln_b.reshape(1, D))


def _sup_body(diag_ref, right_ref, out_ref):
    a = diag_ref[0]
    bmat = right_ref[0]
    lr = lax.broadcasted_iota(jnp.int32, (TE, TE), 0)
    lc = lax.broadcasted_iota(jnp.int32, (TE, TE), 1)
    u1 = jnp.sum(jnp.where(lr == lc - 1, a, 0.0), axis=0)
    u1 = u1.reshape(1, TE)
    u1s = jnp.concatenate([u1[:, 1:], jnp.zeros((1, 1), jnp.float32)], axis=1)
    u2 = jnp.sum(jnp.where(lr == TE - 1, bmat, 0.0), axis=0).reshape(1, TE)
    u2r = jnp.concatenate([jnp.zeros((1, TE - 1), jnp.float32), u2[:, :1]],
                          axis=1)
    out_ref[0] = u1s + u2r


def _prior_sup(prior_A):
    return pl.pallas_call(
        _sup_body,
        grid=(B, NE),
        in_specs=[
            pl.BlockSpec((1, TE, TE), lambda b, r: (b, r, r)),
            pl.BlockSpec((1, TE, TE),
                         lambda b, r: (b, r, jnp.minimum(r + 1, NE - 1))),
        ],
        out_specs=pl.BlockSpec((1, 1, TE), lambda b, r: (b, 0, r)),
        out_shape=jax.ShapeDtypeStruct((B, 1, S_FULL), jnp.float32),
    )(prior_A, prior_A)


def _assemble_body(adiag_ref, loc_ref, amask_ref, psup_ref,
                   wl_ref, wr_ref, pl_ref, pr_ref):
    adiag = adiag_ref[0]
    loc = loc_ref[0]
    i_col = lax.broadcasted_iota(jnp.int32, (S, 1), 0)
    loc_next = jnp.concatenate([loc[1:], loc[-1:]], axis=0)
    keep = ((loc != loc_next) | (i_col == S - 2)) & (i_col <= S - 2)
    j_row = lax.broadcasted_iota(jnp.int32, (S, S_FULL), 1)
    cmp = (loc == j_row) & keep
    val = jnp.sum(jnp.where(cmp, adiag, 0.0), axis=0).reshape(1, S_FULL)
    hit = jnp.max(jnp.where(cmp, 1.0, 0.0), axis=0).reshape(1, S_FULL)
    w = jnp.where(hit > 0.0, val, amask_ref[0])
    psup = psup_ref[0]
    a_sup = psup + (1.0 - psup) * w
    lane = lax.broadcasted_iota(jnp.int32, (1, S_FULL), 1)
    t = jnp.where(lane <= S_FULL - 2, jnp.log(a_sup + EPSILON), 0.0)
    sh = 1
    while sh < S_FULL:
        t = t + jnp.concatenate(
            [jnp.zeros((1, sh), jnp.float32), t[:, : S_FULL - sh]], axis=1)
        sh *= 2
    p = jnp.concatenate(
        [jnp.zeros((1, 1), jnp.float32), t[:, : S_FULL - 1]], axis=1)
    wl_ref[0] = w
    pl_ref[0] = p
    wr_ref[0] = jnp.transpose(w)
    pr_ref[0] = jnp.transpose(p)


def _assemble(adiag, loc_col, amask3, psup3):
    return pl.pallas_call(
        _assemble_body,
        grid=(B,),
        in_specs=[
            pl.BlockSpec((1, S, 1), lambda b: (b, 0, 0)),
            pl.BlockSpec((1, S, 1), lambda b: (b, 0, 0)),
            pl.BlockSpec((1, 1, S_FULL), lambda b: (b, 0, 0)),
            pl.BlockSpec((1, 1, S_FULL), lambda b: (b, 0, 0)),
        ],
        out_specs=[
            pl.BlockSpec((1, 1, S_FULL), lambda b: (b, 0, 0)),
            pl.BlockSpec((1, S_FULL, 1), lambda b: (b, 0, 0)),
            pl.BlockSpec((1, 1, S_FULL), lambda b: (b, 0, 0)),
            pl.BlockSpec((1, S_FULL, 1), lambda b: (b, 0, 0)),
        ],
        out_shape=[
            jax.ShapeDtypeStruct((B, 1, S_FULL), jnp.float32),
            jax.ShapeDtypeStruct((B, S_FULL, 1), jnp.float32),
            jax.ShapeDtypeStruct((B, 1, S_FULL), jnp.float32),
            jax.ShapeDtypeStruct((B, S_FULL, 1), jnp.float32),
        ],
    )(adiag, loc_col, amask3, psup3)


def _big_body(prior_ref, wr_ref, wc_ref, pr_ref, pc_ref, anew_ref, c_ref):
    prior = prior_ref[0]
    r = pl.program_id(1)
    c = pl.program_id(2)
    p_row = pr_ref[0]
    p_col = pc_ref[0, 0].reshape(1, T)

    @pl.when(r == c)
    def _diag():
        row = lax.broadcasted_iota(jnp.int32, (T, T), 0)
        col = lax.broadcasted_iota(jnp.int32, (T, T), 1)
        w_row = wr_ref[0]
        w_col = wc_ref[0, 0].reshape(1, T)
        m = jnp.where(col == row + 1, w_row,
                      jnp.where(row == col + 1, w_col, 0.0))
        anew_ref[0] = prior + (1.0 - prior) * m
        delta = jnp.where(col >= row, p_col - p_row, p_row - p_col)
        c_ref[0] = jnp.exp(delta)

    @pl.when(r != c)
    def _offdiag():
        sgn = jnp.where(c > r, 1.0, -1.0)
        c_ref[0] = jnp.exp((p_col - p_row) * sgn)
        anew_ref[0] = prior

        lane = lax.broadcasted_iota(jnp.int32, (1, T), 1)

        @pl.when(c == r + 1)
        def _fix_up():
            prow = prior[T - 1:T, :]
            wv = wr_ref[0][T - 1:T, :]
            fixed = prow + (1.0 - prow) * wv
            anew_ref[0, T - 1:T, :] = jnp.where(lane == 0, fixed, prow)

        @pl.when(c == r - 1)
        def _fix_lo():
            prow = prior[0:1, :]
            wv = wc_ref[0][:, T - 1:T]
            fixed = prow + (1.0 - prow) * wv
            anew_ref[0, 0:1, :] = jnp.where(lane == T - 1, fixed, prow)


def _big_outputs(prior_A, w_lane, w_rowv, p_lane, p_rowv):
    return pl.pallas_call(
        _big_body,
        grid=(B, NT, NT),
        in_specs=[
            pl.BlockSpec((1, T, T), lambda b, r, c: (b, r, c)),
            pl.BlockSpec((1, T, 1), lambda b, r, c: (b, r, 0)),
            pl.BlockSpec((1, 1, T), lambda b, r, c: (b, 0, c)),
            pl.BlockSpec((1, T, 1), lambda b, r, c: (b, r, 0)),
            pl.BlockSpec((1, 1, T), lambda b, r, c: (b, 0, c)),
        ],
        out_specs=[
            pl.BlockSpec((1, T, T), lambda b, r, c: (b, r, c)),
            pl.BlockSpec((1, T, T), lambda b, r, c: (b, r, c)),
        ],
        out_shape=[
            jax.ShapeDtypeStruct((B, S_FULL, S_FULL), jnp.float32),
            jax.ShapeDtypeStruct((B, S_FULL, S_FULL), jnp.float32),
        ],
        compiler_params=pltpu.CompilerParams(
            dimension_semantics=("parallel", "parallel", "parallel")),
    )(prior_A, w_rowv, w_lane, p_rowv, p_lane)


def kernel(hidden_states, attention_mask, ip_event_loc, ip_event_mask,
           prior_A, Wk, bk, Wq, bq, ln_w, ln_b):
    loc = ip_event_loc.astype(jnp.int32)
    flat_loc = loc.reshape(-1)

    gathered = _gather_rows(hidden_states, flat_loc)
    adiag = _qk_adiag(gathered, Wq, bq, Wk, bk, ln_w, ln_b)
    psup3 = _prior_sup(prior_A)
    amask3 = attention_mask.astype(jnp.float32).reshape(B, 1, S_FULL)
    loc_col = loc.reshape(B, S, 1)
    w_lane, w_rowv, p_lane, p_rowv = _assemble(
        adiag, loc_col, amask3, psup3)
    anew, c_prior = _big_outputs(prior_A, w_lane, w_rowv, p_lane, p_rowv)
    return (c_prior.astype(jnp.float32), anew.astype(jnp.float32))


_PROBE = 0

if _PROBE == 1:
    _full = kernel
    def kernel(hidden_states, attention_mask, ip_event_loc, ip_event_mask,
               prior_A, Wk, bk, Wq, bq, ln_w, ln_b):
        z_lane = jnp.zeros((B, 1, S_FULL), jnp.float32)
        z_row = jnp.zeros((B, S_FULL, 1), jnp.float32)
        anew, c_prior = _big_outputs(prior_A, z_lane, z_row, z_lane, z_row)
        return (c_prior, anew)
elif _PROBE == 2:
    _full = kernel
    def kernel(hidden_states, attention_mask, ip_event_loc, ip_event_mask,
               prior_A, Wk, bk, Wq, bq, ln_w, ln_b):
        loc = ip_event_loc.astype(jnp.int32)
        flat_loc = loc.reshape(-1)
        gathered = _gather_rows(hidden_states, flat_loc)
        adiag = _qk_adiag(gathered, Wq, bq, Wk, bk, ln_w, ln_b)
        psup3 = _prior_sup(prior_A)
        amask3 = attention_mask.astype(jnp.float32).reshape(B, 1, S_FULL)
        loc_col = loc.reshape(B, S, 1)
        w_lane, w_rowv, p_lane, p_rowv = _assemble(adiag, loc_col, amask3, psup3)
        return (w_lane, p_lane)
elif _PROBE == 3:
    _full = kernel
    def kernel(hidden_states, attention_mask, ip_event_loc, ip_event_mask,
               prior_A, Wk, bk, Wq, bq, ln_w, ln_b):
        loc = ip_event_loc.astype(jnp.int32)
        flat_loc = loc.reshape(-1)
        gathered = _gather_rows(hidden_states, flat_loc)
        adiag = _qk_adiag(gathered, Wq, bq, Wk, bk, ln_w, ln_b)
        return (adiag,)
elif _PROBE == 4:
    _full = kernel
    def kernel(hidden_states, attention_mask, ip_event_loc, ip_event_mask,
               prior_A, Wk, bk, Wq, bq, ln_w, ln_b):
        loc = ip_event_loc.astype(jnp.int32)
        flat_loc = loc.reshape(-1)
        gathered = _gather_rows(hidden_states, flat_loc)
        return (gathered,)


# speedup vs baseline: 1.3810x; 1.0674x over previous
import functools

import jax
import jax.numpy as jnp
from jax import lax
from jax.experimental import pallas as pl
from jax.experimental.pallas import tpu as pltpu

EPSILON = 1e-15
LN_EPS = 1e-12
B, S_FULL, S, D = 2, 2048, 512, 1024
G = 8
T = 1024
NT = S_FULL // T
TE = 128
NE = S_FULL // TE


def _gather_body(idx_ref, *refs):
    out_ref = refs[G]
    b = pl.program_id(0)
    j = pl.program_id(1)
    rows = []
    for g in range(G):
        r = idx_ref[b * S + j * G + g] % 8
        rows.append(refs[g][0, pl.ds(r, 1), :])
    out_ref[0] = jnp.concatenate(rows, axis=0)


def _gather_rows(hidden_states, flat_loc):
    grid = (B, S // G)
    in_specs = [
        pl.BlockSpec((1, 8, D), functools.partial(
            lambda g, b, j, idx: (b, idx[b * S + j * G + g] // 8, 0), g))
        for g in range(G)
    ]
    out_spec = pl.BlockSpec((1, G, D), lambda b, j, idx: (b, j, 0))
    return pl.pallas_call(
        _gather_body,
        grid_spec=pltpu.PrefetchScalarGridSpec(
            num_scalar_prefetch=1,
            grid=grid,
            in_specs=in_specs,
            out_specs=out_spec,
        ),
        out_shape=jax.ShapeDtypeStruct((B, S, D), jnp.float32),
    )(flat_loc, *([hidden_states] * G))


def _qk_body(ctx_ref, wq_ref, bq_ref, wk_ref, bk_ref, lnw_ref, lnb_ref, out_ref):
    x = ctx_ref[0]
    mu = jnp.mean(x, axis=1, keepdims=True)
    var = jnp.mean((x - mu) ** 2, axis=1, keepdims=True)
    ctx = (x - mu) / jnp.sqrt(var + LN_EPS) * lnw_ref[...] + lnb_ref[...]
    q = lax.dot_general(ctx, wq_ref[...], (((1,), (1,)), ((), ())),
                        preferred_element_type=jnp.float32) + bq_ref[...]
    k = lax.dot_general(ctx, wk_ref[...], (((1,), (1,)), ((), ())),
                        preferred_element_type=jnp.float32) + bk_ref[...]
    zrow = jnp.zeros((1, D), jnp.float32)
    k_next = jnp.concatenate([k[1:], zrow], axis=0)
    q_next = jnp.concatenate([q[1:], zrow], axis=0)
    scale = D / 2.0
    f = jnp.sum(q * k_next, axis=1, keepdims=True) / scale
    g = jnp.sum(q_next * k, axis=1, keepdims=True) / scale
    zc = jnp.zeros((1, 1), jnp.float32)
    g_prev = jnp.concatenate([zc, g[:-1]], axis=0)
    f_next = jnp.concatenate([f[1:], zc], axis=0)

    def two_sm(a, b):
        m = jnp.maximum(a, b)
        ea = jnp.exp(a - m)
        eb = jnp.exp(b - m)
        return ea / (ea + eb)

    i_col = lax.broadcasted_iota(jnp.int32, (S, 1), 0)
    p = jnp.where(i_col == 0, 1.0, two_sm(f, g_prev))
    r = jnp.where(i_col == S - 2, 1.0, two_sm(g, f_next))
    a_diag = jnp.sqrt(p * r + EPSILON)
    out_ref[0] = jnp.where(i_col <= S - 2, a_diag, 0.0)


def _qk_adiag(ctx, Wq, bq, Wk, bk, ln_w, ln_b):
    return pl.pallas_call(
        _qk_body,
        grid=(B,),
        in_specs=[
            pl.BlockSpec((1, S, D), lambda b: (b, 0, 0)),
            pl.BlockSpec((D, D), lambda b: (0, 0)),
            pl.BlockSpec((1, D), lambda b: (0, 0)),
            pl.BlockSpec((D, D), lambda b: (0, 0)),
            pl.BlockSpec((1, D), lambda b: (0, 0)),
            pl.BlockSpec((1, D), lambda b: (0, 0)),
            pl.BlockSpec((1, D), lambda b: (0, 0)),
        ],
        out_specs=pl.BlockSpec((1, S, 1), lambda b: (b, 0, 0)),
        out_shape=jax.ShapeDtypeStruct((B, S, 1), jnp.float32),
    )(ctx, Wq, bq.reshape(1, D), Wk, bk.reshape(1, D),
      ln_w.reshape(1, D), ln_b.reshape(1, D))


def _sup_body(diag_ref, right_ref, out_ref):
    a = diag_ref[0]
    bmat = right_ref[0]
    lr = lax.broadcasted_iota(jnp.int32, (TE, TE), 0)
    lc = lax.broadcasted_iota(jnp.int32, (TE, TE), 1)
    u1 = jnp.sum(jnp.where(lr == lc - 1, a, 0.0), axis=0)
    u1 = u1.reshape(1, TE)
    u1s = jnp.concatenate([u1[:, 1:], jnp.zeros((1, 1), jnp.float32)], axis=1)
    u2 = jnp.sum(jnp.where(lr == TE - 1, bmat, 0.0), axis=0).reshape(1, TE)
    u2r = jnp.concatenate([jnp.zeros((1, TE - 1), jnp.float32), u2[:, :1]],
                          axis=1)
    out_ref[0] = u1s + u2r


def _prior_sup(prior_A):
    return pl.pallas_call(
        _sup_body,
        grid=(B, NE),
        in_specs=[
            pl.BlockSpec((1, TE, TE), lambda b, r: (b, r, r)),
            pl.BlockSpec((1, TE, TE),
                         lambda b, r: (b, r, jnp.minimum(r + 1, NE - 1))),
        ],
        out_specs=pl.BlockSpec((1, 1, TE), lambda b, r: (b, 0, r)),
        out_shape=jax.ShapeDtypeStruct((B, 1, S_FULL), jnp.float32),
    )(prior_A, prior_A)


def _assemble_body(adiag_ref, loc_ref, amask_ref, psup_ref,
                   wl_ref, wr_ref, pl_ref, pr_ref):
    adiag = adiag_ref[0]
    loc = loc_ref[0]
    i_col = lax.broadcasted_iota(jnp.int32, (S, 1), 0)
    loc_next = jnp.concatenate([loc[1:], loc[-1:]], axis=0)
    keep = ((loc != loc_next) | (i_col == S - 2)) & (i_col <= S - 2)
    j_row = lax.broadcasted_iota(jnp.int32, (S, S_FULL), 1)
    cmp = (loc == j_row) & keep
    val = jnp.sum(jnp.where(cmp, adiag, 0.0), axis=0).reshape(1, S_FULL)
    hit = jnp.max(jnp.where(cmp, 1.0, 0.0), axis=0).reshape(1, S_FULL)
    w = jnp.where(hit > 0.0, val, amask_ref[0])
    psup = psup_ref[0]
    a_sup = psup + (1.0 - psup) * w
    lane = lax.broadcasted_iota(jnp.int32, (1, S_FULL), 1)
    t = jnp.where(lane <= S_FULL - 2, jnp.log(a_sup + EPSILON), 0.0)
    sh = 1
    while sh < S_FULL:
        t = t + jnp.concatenate(
            [jnp.zeros((1, sh), jnp.float32), t[:, : S_FULL - sh]], axis=1)
        sh *= 2
    p = jnp.concatenate(
        [jnp.zeros((1, 1), jnp.float32), t[:, : S_FULL - 1]], axis=1)
    wl_ref[0] = w
    pl_ref[0] = p
    wr_ref[0] = jnp.transpose(w)
    pr_ref[0] = jnp.transpose(p)


def _assemble(adiag, loc_col, amask3, psup3):
    return pl.pallas_call(
        _assemble_body,
        grid=(B,),
        in_specs=[
            pl.BlockSpec((1, S, 1), lambda b: (b, 0, 0)),
            pl.BlockSpec((1, S, 1), lambda b: (b, 0, 0)),
            pl.BlockSpec((1, 1, S_FULL), lambda b: (b, 0, 0)),
            pl.BlockSpec((1, 1, S_FULL), lambda b: (b, 0, 0)),
        ],
        out_specs=[
            pl.BlockSpec((1, 1, S_FULL), lambda b: (b, 0, 0)),
            pl.BlockSpec((1, S_FULL, 1), lambda b: (b, 0, 0)),
            pl.BlockSpec((1, 1, S_FULL), lambda b: (b, 0, 0)),
            pl.BlockSpec((1, S_FULL, 1), lambda b: (b, 0, 0)),
        ],
        out_shape=[
            jax.ShapeDtypeStruct((B, 1, S_FULL), jnp.float32),
            jax.ShapeDtypeStruct((B, S_FULL, 1), jnp.float32),
            jax.ShapeDtypeStruct((B, 1, S_FULL), jnp.float32),
            jax.ShapeDtypeStruct((B, S_FULL, 1), jnp.float32),
        ],
    )(adiag, loc_col, amask3, psup3)


def _big_body(prior_ref, wr_ref, wc_ref, pr_ref, pc_ref, anew_ref, c_ref):
    prior = prior_ref[0]
    r = pl.program_id(1)
    c = pl.program_id(2)
    p_row = pr_ref[0]
    p_col = pc_ref[0, 0].reshape(1, T)

    @pl.when(r == c)
    def _diag():
        row = lax.broadcasted_iota(jnp.int32, (T, T), 0)
        col = lax.broadcasted_iota(jnp.int32, (T, T), 1)
        w_row = wr_ref[0]
        w_col = wc_ref[0, 0].reshape(1, T)
        m = jnp.where(col == row + 1, w_row,
                      jnp.where(row == col + 1, w_col, 0.0))
        anew_ref[0] = prior + (1.0 - prior) * m
        delta = jnp.where(col >= row, p_col - p_row, p_row - p_col)
        c_ref[0] = jnp.exp(delta)

    @pl.when(r != c)
    def _offdiag():
        sgn = jnp.where(c > r, 1.0, -1.0)
        c_ref[0] = jnp.exp((p_col - p_row) * sgn)
        anew_ref[0] = prior

        lane = lax.broadcasted_iota(jnp.int32, (1, T), 1)

        @pl.when(c == r + 1)
        def _fix_up():
            prow = prior[T - 1:T, :]
            wv = wr_ref[0][T - 1:T, :]
            fixed = prow + (1.0 - prow) * wv
            anew_ref[0, T - 1:T, :] = jnp.where(lane == 0, fixed, prow)

        @pl.when(c == r - 1)
        def _fix_lo():
            prow = prior[0:1, :]
            wv = wc_ref[0][:, T - 1:T]
            fixed = prow + (1.0 - prow) * wv
            anew_ref[0, 0:1, :] = jnp.where(lane == T - 1, fixed, prow)


def _big_outputs(prior_A, w_lane, w_rowv, p_lane, p_rowv):
    return pl.pallas_call(
        _big_body,
        grid=(B, NT, NT),
        in_specs=[
            pl.BlockSpec((1, T, T), lambda b, r, c: (b, r, c)),
            pl.BlockSpec((1, T, 1), lambda b, r, c: (b, r, 0)),
            pl.BlockSpec((1, 1, T), lambda b, r, c: (b, 0, c)),
            pl.BlockSpec((1, T, 1), lambda b, r, c: (b, r, 0)),
            pl.BlockSpec((1, 1, T), lambda b, r, c: (b, 0, c)),
        ],
        out_specs=[
            pl.BlockSpec((1, T, T), lambda b, r, c: (b, r, c)),
            pl.BlockSpec((1, T, T), lambda b, r, c: (b, r, c)),
        ],
        out_shape=[
            jax.ShapeDtypeStruct((B, S_FULL, S_FULL), jnp.float32),
            jax.ShapeDtypeStruct((B, S_FULL, S_FULL), jnp.float32),
        ],
        compiler_params=pltpu.CompilerParams(
            dimension_semantics=("parallel", "parallel", "parallel")),
    )(prior_A, w_rowv, w_lane, p_rowv, p_lane)


def kernel(hidden_states, attention_mask, ip_event_loc, ip_event_mask,
           prior_A, Wk, bk, Wq, bq, ln_w, ln_b):
    loc = ip_event_loc.astype(jnp.int32)
    flat_loc = loc.reshape(-1)

    gathered = _gather_rows(hidden_states, flat_loc)
    adiag = _qk_adiag(gathered, Wq, bq, Wk, bk, ln_w, ln_b)
    psup3 = _prior_sup(prior_A)
    amask3 = attention_mask.astype(jnp.float32).reshape(B, 1, S_FULL)
    loc_col = loc.reshape(B, S, 1)
    w_lane, w_rowv, p_lane, p_rowv = _assemble(
        adiag, loc_col, amask3, psup3)
    anew, c_prior = _big_outputs(prior_A, w_lane, w_rowv, p_lane, p_rowv)
    return (c_prior.astype(jnp.float32), anew.astype(jnp.float32))


_PROBE = 0

if _PROBE == 1:
    _full = kernel
    def kernel(hidden_states, attention_mask, ip_event_loc, ip_event_mask,
               prior_A, Wk, bk, Wq, bq, ln_w, ln_b):
        z_lane = jnp.zeros((B, 1, S_FULL), jnp.float32)
        z_row = jnp.zeros((B, S_FULL, 1), jnp.float32)
        anew, c_prior = _big_outputs(prior_A, z_lane, z_row, z_lane, z_row)
        return (c_prior, anew)
elif _PROBE == 2:
    _full = kernel
    def kernel(hidden_states, attention_mask, ip_event_loc, ip_event_mask,
               prior_A, Wk, bk, Wq, bq, ln_w, ln_b):
        loc = ip_event_loc.astype(jnp.int32)
        flat_loc = loc.reshape(-1)
        gathered = _gather_rows(hidden_states, flat_loc)
        adiag = _qk_adiag(gathered, Wq, bq, Wk, bk, ln_w, ln_b)
        psup3 = _prior_sup(prior_A)
        amask3 = attention_mask.astype(jnp.float32).reshape(B, 1, S_FULL)
        loc_col = loc.reshape(B, S, 1)
        w_lane, w_rowv, p_lane, p_rowv = _assemble(adiag, loc_col, amask3, psup3)
        return (w_lane, p_lane)
elif _PROBE == 3:
    _full = kernel
    def kernel(hidden_states, attention_mask, ip_event_loc, ip_event_mask,
               prior_A, Wk, bk, Wq, bq, ln_w, ln_b):
        loc = ip_event_loc.astype(jnp.int32)
        flat_loc = loc.reshape(-1)
        gathered = _gather_rows(hidden_states, flat_loc)
        adiag = _qk_adiag(gathered, Wq, bq, Wk, bk, ln_w, ln_b)
        return (adiag,)
elif _PROBE == 4:
    _full = kernel
    def kernel(hidden_states, attention_mask, ip_event_loc, ip_event_mask,
               prior_A, Wk, bk, Wq, bq, ln_w, ln_b):
        loc = ip_event_loc.astype(jnp.int32)
        flat_loc = loc.reshape(-1)
        gathered = _gather_rows(hidden_states, flat_loc)
        return (gathered,)


# speedup vs baseline: 2.3894x; 1.7302x over previous
import functools

import jax
import jax.numpy as jnp
from jax import lax
from jax.experimental import pallas as pl
from jax.experimental.pallas import tpu as pltpu
from jax.experimental.pallas import tpu_sc as plsc

EPSILON = 1e-15
LN_EPS = 1e-12
B, S_FULL, S, D = 2, 2048, 512, 1024
G = 8
T = 1024
NT = S_FULL // T
TE = 128
NE = S_FULL // TE


def _gather_body(idx_ref, *refs):
    out_ref = refs[G]
    b = pl.program_id(0)
    j = pl.program_id(1)
    rows = []
    for g in range(G):
        r = idx_ref[b * S + j * G + g] % 8
        rows.append(refs[g][0, pl.ds(r, 1), :])
    out_ref[0] = jnp.concatenate(rows, axis=0)


def _gather_rows(hidden_states, flat_loc):
    grid = (B, S // G)
    in_specs = [
        pl.BlockSpec((1, 8, D), functools.partial(
            lambda g, b, j, idx: (b, idx[b * S + j * G + g] // 8, 0), g))
        for g in range(G)
    ]
    out_spec = pl.BlockSpec((1, G, D), lambda b, j, idx: (b, j, 0))
    return pl.pallas_call(
        _gather_body,
        grid_spec=pltpu.PrefetchScalarGridSpec(
            num_scalar_prefetch=1,
            grid=grid,
            in_specs=in_specs,
            out_specs=out_spec,
        ),
        out_shape=jax.ShapeDtypeStruct((B, S, D), jnp.float32),
    )(flat_loc, *([hidden_states] * G))


def _gather_rows_sc(hidden2, flat_loc):
    info = plsc.get_sparse_core_info()
    nw = info.num_cores * info.num_subcores
    rows = B * S
    b_per_w = rows // nw
    mesh = plsc.VectorSubcoreMesh(core_axis_name="c", subcore_axis_name="s")

    @functools.partial(
        pl.kernel, mesh=mesh,
        out_type=jax.ShapeDtypeStruct((rows, D), jnp.float32),
        scratch_types=[
            pltpu.VMEM((b_per_w,), jnp.int32),
            pltpu.VMEM((b_per_w, D), jnp.float32),
            pltpu.SemaphoreType.DMA,
        ],
    )
    def k(table_hbm, idx_hbm, out_hbm, idx_v, rows_v, sem):
        wid = lax.axis_index("s") * info.num_cores + lax.axis_index("c")
        base = wid * b_per_w
        pltpu.sync_copy(idx_hbm.at[pl.ds(base, b_per_w)], idx_v)
        pltpu.async_copy(table_hbm.at[idx_v], rows_v, sem).wait()
        pltpu.sync_copy(rows_v, out_hbm.at[pl.ds(base, b_per_w)])

    return k(hidden2, flat_loc)


def _qk_body(ctx_ref, wq_ref, bq_ref, wk_ref, bk_ref, lnw_ref, lnb_ref, out_ref):
    x = ctx_ref[0]
    mu = jnp.mean(x, axis=1, keepdims=True)
    var = jnp.mean((x - mu) ** 2, axis=1, keepdims=True)
    ctx = (x - mu) / jnp.sqrt(var + LN_EPS) * lnw_ref[...] + lnb_ref[...]
    q = lax.dot_general(ctx, wq_ref[...], (((1,), (1,)), ((), ())),
                        preferred_element_type=jnp.float32) + bq_ref[...]
    k = lax.dot_general(ctx, wk_ref[...], (((1,), (1,)), ((), ())),
                        preferred_element_type=jnp.float32) + bk_ref[...]
    zrow = jnp.zeros((1, D), jnp.float32)
    k_next = jnp.concatenate([k[1:], zrow], axis=0)
    q_next = jnp.concatenate([q[1:], zrow], axis=0)
    scale = D / 2.0
    f = jnp.sum(q * k_next, axis=1, keepdims=True) / scale
    g = jnp.sum(q_next * k, axis=1, keepdims=True) / scale
    zc = jnp.zeros((1, 1), jnp.float32)
    g_prev = jnp.concatenate([zc, g[:-1]], axis=0)
    f_next = jnp.concatenate([f[1:], zc], axis=0)

    def two_sm(a, b):
        m = jnp.maximum(a, b)
        ea = jnp.exp(a - m)
        eb = jnp.exp(b - m)
        return ea / (ea + eb)

    i_col = lax.broadcasted_iota(jnp.int32, (S, 1), 0)
    p = jnp.where(i_col == 0, 1.0, two_sm(f, g_prev))
    r = jnp.where(i_col == S - 2, 1.0, two_sm(g, f_next))
    a_diag = jnp.sqrt(p * r + EPSILON)
    out_ref[0] = jnp.where(i_col <= S - 2, a_diag, 0.0)


def _qk_adiag(ctx, Wq, bq, Wk, bk, ln_w, ln_b):
    return pl.pallas_call(
        _qk_body,
        grid=(B,),
        in_specs=[
            pl.BlockSpec((1, S, D), lambda b: (b, 0, 0)),
            pl.BlockSpec((D, D), lambda b: (0, 0)),
            pl.BlockSpec((1, D), lambda b: (0, 0)),
            pl.BlockSpec((D, D), lambda b: (0, 0)),
            pl.BlockSpec((1, D), lambda b: (0, 0)),
            pl.BlockSpec((1, D), lambda b: (0, 0)),
            pl.BlockSpec((1, D), lambda b: (0, 0)),
        ],
        out_specs=pl.BlockSpec((1, S, 1), lambda b: (b, 0, 0)),
        out_shape=jax.ShapeDtypeStruct((B, S, 1), jnp.float32),
    )(ctx, Wq, bq.reshape(1, D), Wk, bk.reshape(1, D),
      ln_w.reshape(1, D), ln_b.reshape(1, D))


def _sup_body(diag_ref, right_ref, out_ref):
    a = diag_ref[0]
    bmat = right_ref[0]
    lr = lax.broadcasted_iota(jnp.int32, (TE, TE), 0)
    lc = lax.broadcasted_iota(jnp.int32, (TE, TE), 1)
    u1 = jnp.sum(jnp.where(lr == lc - 1, a, 0.0), axis=0)
    u1 = u1.reshape(1, TE)
    u1s = jnp.concatenate([u1[:, 1:], jnp.zeros((1, 1), jnp.float32)], axis=1)
    u2 = jnp.sum(jnp.where(lr == TE - 1, bmat, 0.0), axis=0).reshape(1, TE)
    u2r = jnp.concatenate([jnp.zeros((1, TE - 1), jnp.float32), u2[:, :1]],
                          axis=1)
    out_ref[0] = u1s + u2r


def _prior_sup(prior_A):
    return pl.pallas_call(
        _sup_body,
        grid=(B, NE),
        in_specs=[
            pl.BlockSpec((1, TE, TE), lambda b, r: (b, r, r)),
            pl.BlockSpec((1, TE, TE),
                         lambda b, r: (b, r, jnp.minimum(r + 1, NE - 1))),
        ],
        out_specs=pl.BlockSpec((1, 1, TE), lambda b, r: (b, 0, r)),
        out_shape=jax.ShapeDtypeStruct((B, 1, S_FULL), jnp.float32),
    )(prior_A, prior_A)


def _assemble_body(adiag_ref, loc_ref, amask_ref, psup_ref,
                   wl_ref, wr_ref, pl_ref, pr_ref):
    adiag = adiag_ref[0]
    loc = loc_ref[0]
    i_col = lax.broadcasted_iota(jnp.int32, (S, 1), 0)
    loc_next = jnp.concatenate([loc[1:], loc[-1:]], axis=0)
    keep = ((loc != loc_next) | (i_col == S - 2)) & (i_col <= S - 2)
    j_row = lax.broadcasted_iota(jnp.int32, (S, S_FULL), 1)
    cmp = (loc == j_row) & keep
    val = jnp.sum(jnp.where(cmp, adiag, 0.0), axis=0).reshape(1, S_FULL)
    hit = jnp.max(jnp.where(cmp, 1.0, 0.0), axis=0).reshape(1, S_FULL)
    w = jnp.where(hit > 0.0, val, amask_ref[0])
    psup = psup_ref[0]
    a_sup = psup + (1.0 - psup) * w
    lane = lax.broadcasted_iota(jnp.int32, (1, S_FULL), 1)
    t = jnp.where(lane <= S_FULL - 2, jnp.log(a_sup + EPSILON), 0.0)
    sh = 1
    while sh < S_FULL:
        t = t + jnp.concatenate(
            [jnp.zeros((1, sh), jnp.float32), t[:, : S_FULL - sh]], axis=1)
        sh *= 2
    p = jnp.concatenate(
        [jnp.zeros((1, 1), jnp.float32), t[:, : S_FULL - 1]], axis=1)
    wl_ref[0] = w
    pl_ref[0] = p
    wr_ref[0] = jnp.transpose(w)
    pr_ref[0] = jnp.transpose(p)


def _assemble(adiag, loc_col, amask3, psup3):
    return pl.pallas_call(
        _assemble_body,
        grid=(B,),
        in_specs=[
            pl.BlockSpec((1, S, 1), lambda b: (b, 0, 0)),
            pl.BlockSpec((1, S, 1), lambda b: (b, 0, 0)),
            pl.BlockSpec((1, 1, S_FULL), lambda b: (b, 0, 0)),
            pl.BlockSpec((1, 1, S_FULL), lambda b: (b, 0, 0)),
        ],
        out_specs=[
            pl.BlockSpec((1, 1, S_FULL), lambda b: (b, 0, 0)),
            pl.BlockSpec((1, S_FULL, 1), lambda b: (b, 0, 0)),
            pl.BlockSpec((1, 1, S_FULL), lambda b: (b, 0, 0)),
            pl.BlockSpec((1, S_FULL, 1), lambda b: (b, 0, 0)),
        ],
        out_shape=[
            jax.ShapeDtypeStruct((B, 1, S_FULL), jnp.float32),
            jax.ShapeDtypeStruct((B, S_FULL, 1), jnp.float32),
            jax.ShapeDtypeStruct((B, 1, S_FULL), jnp.float32),
            jax.ShapeDtypeStruct((B, S_FULL, 1), jnp.float32),
        ],
    )(adiag, loc_col, amask3, psup3)


def _big_body(prior_ref, wr_ref, wc_ref, pr_ref, pc_ref, anew_ref, c_ref):
    prior = prior_ref[0]
    r = pl.program_id(1)
    c = pl.program_id(2)
    p_row = pr_ref[0]
    p_col = pc_ref[0, 0].reshape(1, T)

    @pl.when(r == c)
    def _diag():
        row = lax.broadcasted_iota(jnp.int32, (T, T), 0)
        col = lax.broadcasted_iota(jnp.int32, (T, T), 1)
        w_row = wr_ref[0]
        w_col = wc_ref[0, 0].reshape(1, T)
        m = jnp.where(col == row + 1, w_row,
                      jnp.where(row == col + 1, w_col, 0.0))
        anew_ref[0] = prior + (1.0 - prior) * m
        delta = jnp.where(col >= row, p_col - p_row, p_row - p_col)
        c_ref[0] = jnp.exp(delta)

    @pl.when(r != c)
    def _offdiag():
        sgn = jnp.where(c > r, 1.0, -1.0)
        c_ref[0] = jnp.exp((p_col - p_row) * sgn)
        anew_ref[0] = prior

        lane = lax.broadcasted_iota(jnp.int32, (1, T), 1)

        @pl.when(c == r + 1)
        def _fix_up():
            prow = prior[T - 1:T, :]
            wv = wr_ref[0][T - 1:T, :]
            fixed = prow + (1.0 - prow) * wv
            anew_ref[0, T - 1:T, :] = jnp.where(lane == 0, fixed, prow)

        @pl.when(c == r - 1)
        def _fix_lo():
            prow = prior[0:1, :]
            wv = wc_ref[0][:, T - 1:T]
            fixed = prow + (1.0 - prow) * wv
            anew_ref[0, 0:1, :] = jnp.where(lane == T - 1, fixed, prow)


def _big_outputs(prior_A, w_lane, w_rowv, p_lane, p_rowv):
    return pl.pallas_call(
        _big_body,
        grid=(B, NT, NT),
        in_specs=[
            pl.BlockSpec((1, T, T), lambda b, r, c: (b, r, c)),
            pl.BlockSpec((1, T, 1), lambda b, r, c: (b, r, 0)),
            pl.BlockSpec((1, 1, T), lambda b, r, c: (b, 0, c)),
            pl.BlockSpec((1, T, 1), lambda b, r, c: (b, r, 0)),
            pl.BlockSpec((1, 1, T), lambda b, r, c: (b, 0, c)),
        ],
        out_specs=[
            pl.BlockSpec((1, T, T), lambda b, r, c: (b, r, c)),
            pl.BlockSpec((1, T, T), lambda b, r, c: (b, r, c)),
        ],
        out_shape=[
            jax.ShapeDtypeStruct((B, S_FULL, S_FULL), jnp.float32),
            jax.ShapeDtypeStruct((B, S_FULL, S_FULL), jnp.float32),
        ],
        compiler_params=pltpu.CompilerParams(
            dimension_semantics=("parallel", "parallel", "parallel")),
    )(prior_A, w_rowv, w_lane, p_rowv, p_lane)


def kernel(hidden_states, attention_mask, ip_event_loc, ip_event_mask,
           prior_A, Wk, bk, Wq, bq, ln_w, ln_b):
    loc = ip_event_loc.astype(jnp.int32)
    flat_idx = (jnp.arange(B, dtype=jnp.int32)[:, None] * S_FULL
                + loc).reshape(-1)

    hidden2 = hidden_states.reshape(B * S_FULL, D)
    gathered = _gather_rows_sc(hidden2, flat_idx).reshape(B, S, D)
    adiag = _qk_adiag(gathered, Wq, bq, Wk, bk, ln_w, ln_b)
    psup3 = _prior_sup(prior_A)
    amask3 = attention_mask.astype(jnp.float32).reshape(B, 1, S_FULL)
    loc_col = loc.reshape(B, S, 1)
    w_lane, w_rowv, p_lane, p_rowv = _assemble(
        adiag, loc_col, amask3, psup3)
    anew, c_prior = _big_outputs(prior_A, w_lane, w_rowv, p_lane, p_rowv)
    return (c_prior.astype(jnp.float32), anew.astype(jnp.float32))


_PROBE = 0

if _PROBE == 1:
    _full = kernel
    def kernel(hidden_states, attention_mask, ip_event_loc, ip_event_mask,
               prior_A, Wk, bk, Wq, bq, ln_w, ln_b):
        z_lane = jnp.zeros((B, 1, S_FULL), jnp.float32)
        z_row = jnp.zeros((B, S_FULL, 1), jnp.float32)
        anew, c_prior = _big_outputs(prior_A, z_lane, z_row, z_lane, z_row)
        return (c_prior, anew)
elif _PROBE == 2:
    _full = kernel
    def kernel(hidden_states, attention_mask, ip_event_loc, ip_event_mask,
               prior_A, Wk, bk, Wq, bq, ln_w, ln_b):
        loc = ip_event_loc.astype(jnp.int32)
        flat_loc = loc.reshape(-1)
        gathered = _gather_rows(hidden_states, flat_loc)
        adiag = _qk_adiag(gathered, Wq, bq, Wk, bk, ln_w, ln_b)
        psup3 = _prior_sup(prior_A)
        amask3 = attention_mask.astype(jnp.float32).reshape(B, 1, S_FULL)
        loc_col = loc.reshape(B, S, 1)
        w_lane, w_rowv, p_lane, p_rowv = _assemble(adiag, loc_col, amask3, psup3)
        return (w_lane, p_lane)
elif _PROBE == 3:
    _full = kernel
    def kernel(hidden_states, attention_mask, ip_event_loc, ip_event_mask,
               prior_A, Wk, bk, Wq, bq, ln_w, ln_b):
        loc = ip_event_loc.astype(jnp.int32)
        flat_loc = loc.reshape(-1)
        gathered = _gather_rows(hidden_states, flat_loc)
        adiag = _qk_adiag(gathered, Wq, bq, Wk, bk, ln_w, ln_b)
        return (adiag,)
elif _PROBE == 4:
    _full = kernel
    def kernel(hidden_states, attention_mask, ip_event_loc, ip_event_mask,
               prior_A, Wk, bk, Wq, bq, ln_w, ln_b):
        loc = ip_event_loc.astype(jnp.int32)
        flat_loc = loc.reshape(-1)
        gathered = _gather_rows(hidden_states, flat_loc)
        return (gathered,)


# speedup vs baseline: 3.0643x; 1.2824x over previous
import functools

import jax
import jax.numpy as jnp
from jax import lax
from jax.experimental import pallas as pl
from jax.experimental.pallas import tpu as pltpu
from jax.experimental.pallas import tpu_sc as plsc

EPSILON = 1e-15
LN_EPS = 1e-12
B, S_FULL, S, D = 2, 2048, 512, 1024
H = S_FULL // 2


def _gather_rows_sc(hidden2, flat_idx):
    info = plsc.get_sparse_core_info()
    nw = info.num_cores * info.num_subcores
    rows = B * S
    b_per_w = rows // nw
    mesh = plsc.VectorSubcoreMesh(core_axis_name="c", subcore_axis_name="s")

    @functools.partial(
        pl.kernel, mesh=mesh,
        out_type=jax.ShapeDtypeStruct((rows, D), jnp.float32),
        scratch_types=[
            pltpu.VMEM((b_per_w,), jnp.int32),
            pltpu.VMEM((b_per_w, D), jnp.float32),
            pltpu.SemaphoreType.DMA,
        ],
    )
    def k(table_hbm, idx_hbm, out_hbm, idx_v, rows_v, sem):
        wid = lax.axis_index("s") * info.num_cores + lax.axis_index("c")
        base = wid * b_per_w
        pltpu.sync_copy(idx_hbm.at[pl.ds(base, b_per_w)], idx_v)
        pltpu.async_copy(table_hbm.at[idx_v], rows_v, sem).wait()
        pltpu.sync_copy(rows_v, out_hbm.at[pl.ds(base, b_per_w)])

    return k(hidden2, flat_idx)


def _extract_sup(a):
    lr = lax.broadcasted_iota(jnp.int32, (H, H), 0)
    lc = lax.broadcasted_iota(jnp.int32, (H, H), 1)
    u1 = jnp.sum(jnp.where(lr == lc - 1, a, 0.0), axis=0).reshape(1, H)
    return jnp.concatenate([u1[:, 1:], jnp.zeros((1, 1), jnp.float32)],
                           axis=1)


def _excl_cumsum(t):
    sh = 1
    while sh < H:
        t = t + jnp.concatenate(
            [jnp.zeros((1, sh), jnp.float32), t[:, : H - sh]], axis=1)
        sh *= 2
    return jnp.concatenate(
        [jnp.zeros((1, 1), jnp.float32), t[:, : H - 1]], axis=1)


def _mega_body(ctx_ref, wq_ref, bq_ref, wk_ref, bk_ref, lnw_ref, lnb_ref,
               loc_ref, amask_ref, prior_ref, corner_ref,
               anew_ref, c_ref,
               wl_scr, wr_scr, pl_scr, pr_scr):
    s = pl.program_id(1)
    r = jnp.where(s < 2, s, s - 2)
    c = jnp.where(s < 2, s, 3 - s)
    prior = prior_ref[0]

    @pl.when(s == 0)
    def _front():
        x = ctx_ref[0]
        mu = jnp.mean(x, axis=1, keepdims=True)
        var = jnp.mean((x - mu) ** 2, axis=1, keepdims=True)
        ctx = (x - mu) / jnp.sqrt(var + LN_EPS) * lnw_ref[...] + lnb_ref[...]
        q = lax.dot_general(ctx, wq_ref[...], (((1,), (1,)), ((), ())),
                            preferred_element_type=jnp.float32) + bq_ref[...]
        k = lax.dot_general(ctx, wk_ref[...], (((1,), (1,)), ((), ())),
                            preferred_element_type=jnp.float32) + bk_ref[...]
        zrow = jnp.zeros((1, D), jnp.float32)
        k_next = jnp.concatenate([k[1:], zrow], axis=0)
        q_next = jnp.concatenate([q[1:], zrow], axis=0)
        scale = D / 2.0
        f = jnp.sum(q * k_next, axis=1, keepdims=True) / scale
        g = jnp.sum(q_next * k, axis=1, keepdims=True) / scale
        zc = jnp.zeros((1, 1), jnp.float32)
        g_prev = jnp.concatenate([zc, g[:-1]], axis=0)
        f_next = jnp.concatenate([f[1:], zc], axis=0)

        def two_sm(a, b):
            m = jnp.maximum(a, b)
            ea = jnp.exp(a - m)
            eb = jnp.exp(b - m)
            return ea / (ea + eb)

        i_col = lax.broadcasted_iota(jnp.int32, (S, 1), 0)
        p = jnp.where(i_col == 0, 1.0, two_sm(f, g_prev))
        rr = jnp.where(i_col == S - 2, 1.0, two_sm(g, f_next))
        adiag = jnp.sqrt(p * rr + EPSILON)

        loc = loc_ref[0]
        loc_next = jnp.concatenate([loc[1:], loc[-1:]], axis=0)
        keep = ((loc != loc_next) | (i_col == S - 2)) & (i_col <= S - 2)
        j_row = lax.broadcasted_iota(jnp.int32, (S, S_FULL), 1)
        cmp = (loc == j_row) & keep
        val = jnp.sum(jnp.where(cmp, adiag, 0.0), axis=0).reshape(1, S_FULL)
        hit = jnp.max(jnp.where(cmp, 1.0, 0.0), axis=0).reshape(1, S_FULL)
        w = jnp.where(hit > 0.0, val, amask_ref[0])
        wl_scr[...] = w
        wr_scr[...] = jnp.transpose(w)

        sup0 = _extract_sup(prior)
        a_sup0 = sup0 + (1.0 - sup0) * w[:, :H]
        lane = lax.broadcasted_iota(jnp.int32, (1, H), 1)
        t0 = jnp.where(lane <= H - 2, jnp.log(a_sup0 + EPSILON), 0.0)
        p0 = _excl_cumsum(t0)
        pl_scr[0:1, 0:H] = p0
        pr_scr[0:H, :] = jnp.transpose(p0)

    @pl.when(s == 1)
    def _second():
        pcorn = corner_ref[0][7:8, 0:1]
        w_half = wl_scr[0:1, H:]
        w1023 = wl_scr[0:1, H - 1:H]
        t_corner = jnp.log(pcorn + (1.0 - pcorn) * w1023 + EPSILON)
        sup1 = _extract_sup(prior)
        a_sup1 = sup1 + (1.0 - sup1) * w_half
        lane = lax.broadcasted_iota(jnp.int32, (1, H), 1)
        t_tile = jnp.where(lane <= H - 2, jnp.log(a_sup1 + EPSILON), 0.0)
        t1 = jnp.concatenate([t_corner, t_tile[:, : H - 1]], axis=1)
        sh = 1
        inc = t1
        while sh < H:
            inc = inc + jnp.concatenate(
                [jnp.zeros((1, sh), jnp.float32), inc[:, : H - sh]], axis=1)
            sh *= 2
        base = pl_scr[0:1, H - 1:H]
        p1 = base + inc
        pl_scr[0:1, H:] = p1
        pr_scr[H:, :] = jnp.transpose(p1)

    p_row = jnp.where(r == 0, pr_scr[0:H, :], pr_scr[H:, :])
    p_col = jnp.where(c == 0, pl_scr[0:1, 0:H], pl_scr[0:1, H:])

    @pl.when(r == c)
    def _diag():
        row = lax.broadcasted_iota(jnp.int32, (H, H), 0)
        col = lax.broadcasted_iota(jnp.int32, (H, H), 1)
        w_row = jnp.where(r == 0, wr_scr[0:H, :], wr_scr[H:, :])
        w_col = jnp.where(c == 0, wl_scr[0:1, 0:H], wl_scr[0:1, H:])
        m = jnp.where(col == row + 1, w_row,
                      jnp.where(row == col + 1, w_col, 0.0))
        anew_ref[0] = prior + (1.0 - prior) * m
        delta = jnp.where(col >= row, p_col - p_row, p_row - p_col)
        c_ref[0] = jnp.exp(delta)

    @pl.when(r != c)
    def _offdiag():
        sgn = jnp.where(c > r, 1.0, -1.0)
        c_ref[0] = jnp.exp((p_col - p_row) * sgn)
        anew_ref[0] = prior
        lane = lax.broadcasted_iota(jnp.int32, (1, H), 1)
        wmid = wl_scr[0:1, H - 1:H]

        @pl.when(c == r + 1)
        def _fix_up():
            prow = prior[H - 1:H, :]
            fixed = prow + (1.0 - prow) * wmid
            anew_ref[0, H - 1:H, :] = jnp.where(lane == 0, fixed, prow)

        @pl.when(c == r - 1)
        def _fix_lo():
            prow = prior[0:1, :]
            fixed = prow + (1.0 - prow) * wmid
            anew_ref[0, 0:1, :] = jnp.where(lane == H - 1, fixed, prow)


def _mega(ctx, Wq, bq, Wk, bk, ln_w, ln_b, loc_col, amask3, prior_A):
    def rmap(b, s):
        return jnp.where(s < 2, s, s - 2)

    def cmap(b, s):
        return jnp.where(s < 2, s, 3 - s)

    return pl.pallas_call(
        _mega_body,
        grid=(B, 4),
        in_specs=[
            pl.BlockSpec((1, S, D), lambda b, s: (b, 0, 0)),
            pl.BlockSpec((D, D), lambda b, s: (0, 0)),
            pl.BlockSpec((1, D), lambda b, s: (0, 0)),
            pl.BlockSpec((D, D), lambda b, s: (0, 0)),
            pl.BlockSpec((1, D), lambda b, s: (0, 0)),
            pl.BlockSpec((1, D), lambda b, s: (0, 0)),
            pl.BlockSpec((1, D), lambda b, s: (0, 0)),
            pl.BlockSpec((1, S, 1), lambda b, s: (b, 0, 0)),
            pl.BlockSpec((1, 1, S_FULL), lambda b, s: (b, 0, 0)),
            pl.BlockSpec((1, H, H), lambda b, s: (b, rmap(b, s), cmap(b, s))),
            pl.BlockSpec((1, 8, 128), lambda b, s: (b, 127, 8)),
        ],
        out_specs=[
            pl.BlockSpec((1, H, H), lambda b, s: (b, rmap(b, s), cmap(b, s))),
            pl.BlockSpec((1, H, H), lambda b, s: (b, rmap(b, s), cmap(b, s))),
        ],
        out_shape=[
            jax.ShapeDtypeStruct((B, S_FULL, S_FULL), jnp.float32),
            jax.ShapeDtypeStruct((B, S_FULL, S_FULL), jnp.float32),
        ],
        scratch_shapes=[
            pltpu.VMEM((1, S_FULL), jnp.float32),
            pltpu.VMEM((S_FULL, 1), jnp.float32),
            pltpu.VMEM((1, S_FULL), jnp.float32),
            pltpu.VMEM((S_FULL, 1), jnp.float32),
        ],
    )(ctx, Wq, bq.reshape(1, D), Wk, bk.reshape(1, D),
      ln_w.reshape(1, D), ln_b.reshape(1, D), loc_col, amask3, prior_A,
      prior_A)


def kernel(hidden_states, attention_mask, ip_event_loc, ip_event_mask,
           prior_A, Wk, bk, Wq, bq, ln_w, ln_b):
    loc = ip_event_loc.astype(jnp.int32)
    flat_idx = (jnp.arange(B, dtype=jnp.int32)[:, None] * S_FULL
                + loc).reshape(-1)

    hidden2 = hidden_states.reshape(B * S_FULL, D)
    gathered = _gather_rows_sc(hidden2, flat_idx).reshape(B, S, D)
    amask3 = attention_mask.astype(jnp.float32).reshape(B, 1, S_FULL)
    loc_col = loc.reshape(B, S, 1)
    anew, c_prior = _mega(gathered, Wq, bq, Wk, bk, ln_w, ln_b,
                          loc_col, amask3, prior_A)
    return (c_prior.astype(jnp.float32), anew.astype(jnp.float32))
